# Initial kernel scaffold; baseline (speedup 1.0000x reference)
import jax
import jax.numpy as jnp
from jax.experimental import pallas as pl


def kernel(feature, edge_index0, edge_index1, edge_index2, edge_index3, node_types, W1, b1, W2, b2, W3, b3, W4, b4, fc1_w, fc2_w):
    raise NotImplementedError("write your pallas kernel here")


# speedup vs baseline: 15.7153x; 15.7153x over previous
import functools

import jax
import jax.numpy as jnp
from jax import lax
from jax.experimental import pallas as pl
from jax.experimental.pallas import tpu as pltpu
from jax.experimental.pallas import tpu_sc as plsc

N = 10000
E = 320000
NFEAT = 128
OUT = 64
NV = 4

_mesh = lambda: plsc.VectorSubcoreMesh(core_axis_name="c", subcore_axis_name="s")

_E4 = NV * E
_W = 32
_PERW1 = _E4 // _W
_B1 = 2000
_NC1 = _PERW1 // _B1


@functools.partial(
    pl.kernel,
    mesh=_mesh(),
    compiler_params=pltpu.CompilerParams(needs_layout_passes=False),
    out_type=jax.ShapeDtypeStruct((_W * NV * N,), jnp.float32),
    scratch_types=[
        pltpu.VMEM((_B1,), jnp.int32),
        pltpu.VMEM((NV * N,), jnp.float32),
    ],
)
def _k1_degree(cols_hbm, out_hbm, idx_v, deg_v):
    cid = lax.axis_index("c")
    sid = lax.axis_index("s")
    w = cid * 16 + sid

    def zero(k, carry):
        deg_v[pl.ds(k * 16, 16)] = jnp.zeros((16,), jnp.float32)
        return carry

    lax.fori_loop(0, NV * N // 16, zero, 0)

    ones16 = jnp.ones((16,), jnp.float32)
    for j in range(_NC1):
        off = pl.multiple_of(w * _PERW1 + j * _B1, 8)
        pltpu.sync_copy(cols_hbm.at[pl.ds(off, _B1)], idx_v)

        def upd(k, carry):
            i16 = idx_v[pl.ds(k * 16, 16)]
            plsc.addupdate_scatter(deg_v, [i16], ones16)
            return carry

        lax.fori_loop(0, _B1 // 16, upd, 0)

    pltpu.sync_copy(deg_v, out_hbm.at[pl.ds(w * NV * N, NV * N)])


_BN = 400
_GI = N // _BN


def _k2_body(f_ref, w_ref, deg_ref, g_ref, dinv_ref):
    d = jnp.sum(deg_ref[...], axis=1, keepdims=True) + 1.0
    dinv = lax.rsqrt(d)
    h = jax.lax.dot_general(
        f_ref[...], w_ref[...],
        (((1,), (1,)), ((), ())),
        preferred_element_type=jnp.float32,
    )
    g_ref[...] = jnp.concatenate(
        [h * dinv, jnp.zeros((_BN, OUT), jnp.float32)], axis=1)
    dinv_ref[...] = jnp.broadcast_to(dinv, (_BN, 16))


def _k2_transform(feature, wcat, degT):
    return pl.pallas_call(
        _k2_body,
        grid=(NV, _GI),
        in_specs=[
            pl.BlockSpec((_BN, NFEAT), lambda v, i: (i, 0)),
            pl.BlockSpec((OUT, NFEAT), lambda v, i: (v, 0)),
            pl.BlockSpec((_BN, _W), lambda v, i: (v * _GI + i, 0)),
        ],
        out_specs=[
            pl.BlockSpec((_BN, 2 * OUT), lambda v, i: (v * _GI + i, 0)),
            pl.BlockSpec((_BN, 16), lambda v, i: (v * _GI + i, 0)),
        ],
        out_shape=[
            jax.ShapeDtypeStruct((NV * N, 2 * OUT), jnp.float32),
            jax.ShapeDtypeStruct((NV * N, 16), jnp.float32),
        ],
    )(feature, wcat, degT)


_PERT3 = E // 16
_B3 = 400
_NC3 = _PERT3 // _B3
_NH = N // 2


@functools.partial(
    pl.kernel,
    mesh=_mesh(),
    compiler_params=pltpu.CompilerParams(needs_layout_passes=False),
    out_type=jax.ShapeDtypeStruct((NV * N, 2 * OUT), jnp.float32),
    scratch_types=[
        pltpu.VMEM((_B3,), jnp.int32),
        pltpu.VMEM((_B3,), jnp.int32),
        pltpu.VMEM((_B3, 2 * OUT), jnp.float32),
        pltpu.VMEM_SHARED((_NH + 8, 2 * OUT), jnp.float32),
        pltpu.SemaphoreType.DMA,
    ],
)
def _k3_scatter(rows_hbm, cols_hbm, g_hbm, out_hbm,
                ridx, cidx, buf, acc, sem):
    cid = lax.axis_index("c")
    sid = lax.axis_index("s")
    lo = cid * _NH

    dump16 = jnp.full((16,), _NH, jnp.int32)
    lo16 = jnp.full((16,), 1, jnp.int32) * lo

    for v in range(NV):
        def zrow(k, carry):
            buf[k // 8, pl.ds((k % 8) * 16, 16)] = jnp.zeros((16,), jnp.float32)
            return carry

        lax.fori_loop(0, _B3 * 8, zrow, 0)

        @pl.when(sid < 5)
        def _():
            off = pl.multiple_of(sid * 1000, 8)
            pltpu.sync_copy(buf, acc.at[pl.ds(off, _B3)])
            pltpu.sync_copy(buf, acc.at[pl.ds(off + 400, _B3)])
            pltpu.sync_copy(buf.at[pl.ds(0, 200)],
                            acc.at[pl.ds(off + 800, 200)])

        plsc.subcore_barrier()
        for j in range(_NC3):
            off = pl.multiple_of(v * E + sid * _PERT3 + j * _B3, 8)
            pltpu.sync_copy(rows_hbm.at[pl.ds(off, _B3)], ridx)
            pltpu.sync_copy(cols_hbm.at[pl.ds(off, _B3)], cidx)
            pltpu.async_copy(g_hbm.at[ridx], buf, sem).wait()

            def clamp(k, carry):
                i16 = cidx[pl.ds(k * 16, 16)] - lo16
                sel = (i16 >= 0) & (i16 < _NH)
                cidx[pl.ds(k * 16, 16)] = jnp.where(sel, i16, dump16)
                return carry

            lax.fori_loop(0, _B3 // 16, clamp, 0)
            pltpu.sync_copy(buf, acc.at[cidx], add=True)
        plsc.subcore_barrier()

        @pl.when(sid < 5)
        def _():
            off = pl.multiple_of(sid * 1000, 8)
            pltpu.sync_copy(
                acc.at[pl.ds(off, 1000)],
                out_hbm.at[pl.ds(v * N + lo + off, 1000)],
            )
        plsc.subcore_barrier()


def _k4_body(g0, g1, g2, g3, a0, a1, a2, a3, d0, d1, d2, d3, b_ref,
             fc1_ref, fc2_ref, y_ref, sums_ref, betas_ref):
    i = pl.program_id(0)
    gs = (g0, g1, g2, g3)
    accs = (a0, a1, a2, a3)
    ds_ = (d0, d1, d2, d3)
    ys = []
    parts = []
    for v in range(NV):
        pre = ds_[v][:, 0:1] * (gs[v][:, :OUT] + accs[v][:, :OUT]) \
            + b_ref[v:v + 1, :]
        yv = jnp.maximum(pre, 0.0)
        ys.append(yv)
        parts.append(jnp.sum(yv, axis=0, keepdims=True))
    y_ref[...] = jnp.concatenate(ys, axis=1)
    part = jnp.concatenate(parts + [jnp.zeros((4, OUT), jnp.float32)], axis=0)

    @pl.when(i == 0)
    def _():
        sums_ref[...] = part

    @pl.when(i > 0)
    def _():
        sums_ref[...] = sums_ref[...] + part

    @pl.when(i == _GI - 1)
    def _():
        hs = jax.nn.sigmoid(sums_ref[0:4, :] / float(N))
        s0 = jnp.sum(hs * fc1_ref[...], axis=1, keepdims=True)
        s1 = jnp.sum(hs * fc2_ref[...], axis=1, keepdims=True)

        def smax(s):
            e = jnp.exp(s - jnp.max(s, axis=0, keepdims=True))
            return e / jnp.sum(e, axis=0, keepdims=True)

        bb = jnp.concatenate([smax(s0), smax(s1)], axis=0)
        betas_ref[...] = jnp.broadcast_to(bb, (8, 128))


def _k4_fuse(g, acc2, dinv16, bcat, fc1_w, fc2_w):
    vspec = lambda v: pl.BlockSpec((_BN, 2 * OUT), lambda i, v=v: (v * _GI + i, 0))
    dspec = lambda v: pl.BlockSpec((_BN, 16), lambda i, v=v: (v * _GI + i, 0))
    return pl.pallas_call(
        _k4_body,
        grid=(_GI,),
        in_specs=[vspec(0), vspec(1), vspec(2), vspec(3),
                  vspec(0), vspec(1), vspec(2), vspec(3),
                  dspec(0), dspec(1), dspec(2), dspec(3),
                  pl.BlockSpec((NV, OUT), lambda i: (0, 0)),
                  pl.BlockSpec((1, OUT), lambda i: (0, 0)),
                  pl.BlockSpec((1, OUT), lambda i: (0, 0))],
        out_specs=[
            pl.BlockSpec((_BN, NV * OUT), lambda i: (i, 0)),
            pl.BlockSpec((8, OUT), lambda i: (0, 0)),
            pl.BlockSpec((8, 128), lambda i: (0, 0)),
        ],
        out_shape=[
            jax.ShapeDtypeStruct((N, NV * OUT), jnp.float32),
            jax.ShapeDtypeStruct((8, OUT), jnp.float32),
            jax.ShapeDtypeStruct((8, 128), jnp.float32),
        ],
    )(g, g, g, g, acc2, acc2, acc2, acc2, dinv16, dinv16, dinv16, dinv16,
      bcat, fc1_w, fc2_w)


_B5 = 200
_NCH5 = N // _B5


@functools.partial(
    pl.kernel,
    mesh=_mesh(),
    compiler_params=pltpu.CompilerParams(needs_layout_passes=False),
    out_type=jax.ShapeDtypeStruct((N, OUT), jnp.float32),
    scratch_types=[
        pltpu.VMEM((_B5,), jnp.int32),
        pltpu.VMEM((_B5, NV * OUT), jnp.float32),
        pltpu.VMEM((_B5, OUT), jnp.float32),
        pltpu.VMEM((8, 16), jnp.float32),
        pltpu.VMEM((16,), jnp.int32),
        pltpu.SemaphoreType.DMA,
    ],
)
def _k5_mix(y_hbm, perm_hbm, bet_hbm, nz_hbm, out_hbm,
            pidx, ybuf, obuf, betv, nzv, sem):
    cid = lax.axis_index("c")
    sid = lax.axis_index("s")
    w = cid * 16 + sid
    pltpu.sync_copy(bet_hbm, betv)
    pltpu.sync_copy(nz_hbm, nzv)

    def do_chunk(c):
        base = c * _B5
        pltpu.sync_copy(perm_hbm.at[pl.ds(base, _B5)], pidx)
        pltpu.async_copy(y_hbm.at[pidx], ybuf, sem).wait()
        nzvec = nzv[...]

        def row(r, carry):
            rowv = lax.broadcast(base + r, (16,))
            mask = rowv < nzvec
            for cc in range(NV):
                a16 = jnp.zeros((16,), jnp.float32)
                for v in range(NV):
                    bv = jnp.where(mask, betv[v], betv[NV + v])
                    a16 = a16 + bv * ybuf[r, pl.ds(v * OUT + cc * 16, 16)]
                obuf[r, pl.ds(cc * 16, 16)] = a16
            return carry

        lax.fori_loop(0, _B5, row, 0)
        pltpu.sync_copy(obuf, out_hbm.at[pl.ds(base, _B5)])

    do_chunk(w)

    @pl.when(w + _W < _NCH5)
    def _():
        do_chunk(w + _W)


def kernel(feature, edge_index0, edge_index1, edge_index2, edge_index3,
           node_types, W1, b1, W2, b2, W3, b3, W4, b4, fc1_w, fc2_w):
    eis = (edge_index0, edge_index1, edge_index2, edge_index3)
    cols_off = jnp.concatenate(
        [eis[v][1] + jnp.int32(v * N) for v in range(NV)])
    rows_off = jnp.concatenate(
        [eis[v][0] + jnp.int32(v * N) for v in range(NV)])
    cols_cat = jnp.concatenate([eis[v][1] for v in range(NV)])

    deg_flat = _k1_degree(cols_off)
    degT = deg_flat.reshape(_W, NV * N).T

    wcat = jnp.concatenate([W1, W2, W3, W4], axis=0)
    g, dinv16 = _k2_transform(feature, wcat, degT)

    acc2 = _k3_scatter(rows_off, cols_cat, g)

    bcat = jnp.stack([b1, b2, b3, b4], axis=0)
    y, _sums, betas16 = _k4_fuse(g, acc2, dinv16, bcat, fc1_w, fc2_w)

    perm = jnp.argsort(node_types, stable=True).astype(jnp.int32)
    nz = jnp.int32(N) - jnp.sum(node_types, dtype=jnp.int32)
    nz16 = jnp.full((16,), nz, jnp.int32)
    bet16 = betas16[:, :16]

    return _k5_mix(y, perm, bet16, nz16)
